# Initial kernel scaffold; baseline (speedup 1.0000x reference)
import jax
import jax.numpy as jnp
from jax.experimental import pallas as pl


def kernel(word_ids, char_ids, word_table, char_table):
    raise NotImplementedError("write your pallas kernel here")


# speedup vs baseline: 3.6410x; 3.6410x over previous
import functools

import jax
import jax.numpy as jnp
from jax import lax
from jax.experimental import pallas as pl
from jax.experimental.pallas import tpu as pltpu
from jax.experimental.pallas import tpu_sc as plsc


def _word_gather(word_table, idx_flat):
    n = idx_flat.shape[0]
    d = word_table.shape[1]
    window = 128
    mesh = plsc.VectorSubcoreMesh(core_axis_name="core",
                                  subcore_axis_name="subcore")
    idx2 = idx_flat.reshape(1, n)

    @functools.partial(
        pl.kernel,
        out_type=jax.ShapeDtypeStruct((n, d), jnp.float32),
        mesh=mesh,
    )
    def k(tab_hbm, i_hbm, o_hbm):
        def body(i_vmem, o_vmem):
            pltpu.sync_copy(tab_hbm.at[i_vmem.at[0]], o_vmem)

        pltpu.emit_pipeline(
            body,
            grid=(n // window,),
            in_specs=[pl.BlockSpec((1, window), index_map=lambda i: (0, i))],
            out_specs=[pl.BlockSpec((window, d), index_map=lambda i: (i, 0))],
            core_axis_name=("core", "subcore"),
            dimension_semantics=(pltpu.PARALLEL,),
        )(i_hbm, o_hbm)

    return k(word_table, idx2)


def _char_lookup(char_table, idx_flat):
    n = idx_flat.shape[0]
    v, d = char_table.shape
    rows = 1024
    ids2 = idx_flat.reshape(n // 128, 128)

    tab_t = jnp.zeros((d, 128), jnp.float32).at[:, :v].set(char_table.T)

    def body(ids_ref, tab_ref, o_ref):
        ids = ids_ref[...]
        idx = jnp.concatenate(
            [jnp.broadcast_to(ids[s:s + 1, :], (d, 128)) for s in range(8)],
            axis=1)
        out_t = jnp.take_along_axis(tab_ref[...], idx, axis=1)
        o_ref[...] = out_t.T

    return pl.pallas_call(
        body,
        grid=(n // rows,),
        in_specs=[
            pl.BlockSpec((rows // 128, 128), lambda i: (i, 0)),
            pl.BlockSpec((d, 128), lambda i: (0, 0)),
        ],
        out_specs=pl.BlockSpec((rows, d), lambda i: (i, 0)),
        out_shape=jax.ShapeDtypeStruct((n, d), jnp.float32),
    )(ids2, tab_t)


def kernel(word_ids, char_ids, word_table, char_table):
    b, l = word_ids.shape
    w = char_ids.shape[-1]
    wd = word_table.shape[1]
    cd = char_table.shape[1]
    wt_pad = jnp.pad(word_table, ((0, 0), (0, 128 - wd)))
    word_out = _word_gather(wt_pad, word_ids.reshape(-1))[:, :wd]
    char_out = _char_lookup(char_table, char_ids.reshape(-1))
    return (word_out.reshape(b, l, wd), char_out.reshape(b, l, w, cd))


# speedup vs baseline: 4.1263x; 1.1333x over previous
import functools

import jax
import jax.numpy as jnp
from jax import lax
from jax.experimental import pallas as pl
from jax.experimental.pallas import tpu as pltpu
from jax.experimental.pallas import tpu_sc as plsc


def _retile_table(word_table):
    v, d = word_table.shape
    rows = 2048
    grid = (v + rows - 1) // rows

    def body(x_ref, o_ref):
        o_ref[:, :d] = x_ref[...]

    return pl.pallas_call(
        body,
        grid=(grid,),
        in_specs=[pl.BlockSpec((rows, d), lambda i: (i, 0))],
        out_specs=pl.BlockSpec((rows, 128), lambda i: (i, 0)),
        out_shape=jax.ShapeDtypeStruct((grid * rows, 128), jnp.float32),
    )(word_table)


def _slice_reshape(rows128, b, l, d):
    bb = 8

    def body(x_ref, o_ref):
        for j in range(bb):
            o_ref[j] = x_ref[pl.ds(j * l, l), :d]

    return pl.pallas_call(
        body,
        grid=(b // bb,),
        in_specs=[pl.BlockSpec((bb * l, 128), lambda i: (i, 0))],
        out_specs=pl.BlockSpec((bb, l, d), lambda i: (i, 0, 0)),
        out_shape=jax.ShapeDtypeStruct((b, l, d), jnp.float32),
    )(rows128)


def _word_gather(word_table, idx_flat):
    n = idx_flat.shape[0]
    d = word_table.shape[1]
    window = 128
    mesh = plsc.VectorSubcoreMesh(core_axis_name="core",
                                  subcore_axis_name="subcore")
    idx2 = idx_flat.reshape(1, n)

    @functools.partial(
        pl.kernel,
        out_type=jax.ShapeDtypeStruct((n, d), jnp.float32),
        mesh=mesh,
    )
    def k(tab_hbm, i_hbm, o_hbm):
        def body(i_vmem, o_vmem):
            pltpu.sync_copy(tab_hbm.at[i_vmem.at[0]], o_vmem)

        pltpu.emit_pipeline(
            body,
            grid=(n // window,),
            in_specs=[pl.BlockSpec((1, window), index_map=lambda i: (0, i))],
            out_specs=[pl.BlockSpec((window, d), index_map=lambda i: (i, 0))],
            core_axis_name=("core", "subcore"),
            dimension_semantics=(pltpu.PARALLEL,),
        )(i_hbm, o_hbm)

    return k(word_table, idx2)


def _char_lookup(char_table, idx_flat):
    n = idx_flat.shape[0]
    v, d = char_table.shape
    rows = 1024
    ids2 = idx_flat.reshape(n // 128, 128)

    tab_t = jnp.zeros((d, 128), jnp.float32).at[:, :v].set(char_table.T)

    def body(ids_ref, tab_ref, o_ref):
        ids = ids_ref[...]
        idx = jnp.concatenate(
            [jnp.broadcast_to(ids[s:s + 1, :], (d, 128)) for s in range(8)],
            axis=1)
        out_t = jnp.take_along_axis(tab_ref[...], idx, axis=1)
        o_ref[...] = out_t.T

    return pl.pallas_call(
        body,
        grid=(n // rows,),
        in_specs=[
            pl.BlockSpec((rows // 128, 128), lambda i: (i, 0)),
            pl.BlockSpec((d, 128), lambda i: (0, 0)),
        ],
        out_specs=pl.BlockSpec((rows, d), lambda i: (i, 0)),
        out_shape=jax.ShapeDtypeStruct((n, d), jnp.float32),
    )(ids2, tab_t)


def kernel(word_ids, char_ids, word_table, char_table):
    b, l = word_ids.shape
    w = char_ids.shape[-1]
    wd = word_table.shape[1]
    cd = char_table.shape[1]
    wt_pad = _retile_table(word_table)
    word_rows = _word_gather(wt_pad, word_ids.reshape(-1))
    word_out = _slice_reshape(word_rows, b, l, wd)
    char_out = _char_lookup(char_table, char_ids.reshape(-1))
    return (word_out, char_out.reshape(b, l, w, cd))


# speedup vs baseline: 20.7862x; 5.0375x over previous
import functools

import jax
import jax.numpy as jnp
from jax import lax
from jax.experimental import pallas as pl
from jax.experimental.pallas import tpu as pltpu
from jax.experimental.pallas import tpu_sc as plsc


def _retile_table(word_table_t):
    d, v = word_table_t.shape
    cols = 2048
    grid = (v + cols - 1) // cols

    def body(x_ref, o_ref):
        o_ref[:, :d] = x_ref[...].T

    return pl.pallas_call(
        body,
        grid=(grid,),
        in_specs=[pl.BlockSpec((d, cols), lambda i: (0, i))],
        out_specs=pl.BlockSpec((cols, 128), lambda i: (i, 0)),
        out_shape=jax.ShapeDtypeStruct((grid * cols, 128), jnp.float32),
    )(word_table_t)


def _word_gather(word_table, idx_flat):
    n = idx_flat.shape[0]
    d = word_table.shape[1]
    window = 128
    mesh = plsc.VectorSubcoreMesh(core_axis_name="core",
                                  subcore_axis_name="subcore")
    idx2 = idx_flat.reshape(1, n)

    @functools.partial(
        pl.kernel,
        out_type=jax.ShapeDtypeStruct((n, d), jnp.float32),
        mesh=mesh,
    )
    def k(tab_hbm, i_hbm, o_hbm):
        def body(i_vmem, o_vmem):
            pltpu.sync_copy(tab_hbm.at[i_vmem.at[0]], o_vmem)

        pltpu.emit_pipeline(
            body,
            grid=(n // window,),
            in_specs=[pl.BlockSpec((1, window), index_map=lambda i: (0, i))],
            out_specs=[pl.BlockSpec((window, d), index_map=lambda i: (i, 0))],
            core_axis_name=("core", "subcore"),
            dimension_semantics=(pltpu.PARALLEL,),
        )(i_hbm, o_hbm)

    return k(word_table, idx2)


def _word_to_batch_minor(rows128, b, l, d):
    def body(x_ref, o_ref):
        o_ref[0] = x_ref[:, :d].T

    return pl.pallas_call(
        body,
        grid=(l,),
        in_specs=[pl.BlockSpec((b, 128), lambda i: (i, 0))],
        out_specs=pl.BlockSpec((1, d, b), lambda i: (i, 0, 0)),
        out_shape=jax.ShapeDtypeStruct((l, d, b), jnp.float32),
    )(rows128)


def _char_lookup(char_table, ids_t):
    l, w, b = ids_t.shape
    v, d = char_table.shape
    tab_t = jnp.zeros((d, 128), jnp.float32).at[:, :v].set(char_table.T)

    g = 8

    def body(ids_ref, tab_ref, o_ref):
        tab = tab_ref[...]
        for k in range(g):
            for c in range(b // 128):
                idr = ids_ref[k, pl.ds(c * 128, 128)].reshape(1, 128)
                idx = jnp.broadcast_to(idr, (d, 128))
                o_ref[k, :, pl.ds(c * 128, 128)] = jnp.take_along_axis(
                    tab, idx, axis=1)

    out = pl.pallas_call(
        body,
        grid=(l * w // g,),
        in_specs=[
            pl.BlockSpec((g, b), lambda i: (i, 0)),
            pl.BlockSpec((d, 128), lambda i: (0, 0)),
        ],
        out_specs=pl.BlockSpec((g, d, b), lambda i: (i, 0, 0)),
        out_shape=jax.ShapeDtypeStruct((l * w, d, b), jnp.float32),
    )(ids_t.reshape(l * w, b), tab_t)
    return out.reshape(l, w, d, b)


def kernel(word_ids, char_ids, word_table, char_table):
    b, l = word_ids.shape
    w = char_ids.shape[-1]
    wd = word_table.shape[1]

    wt_pad = _retile_table(word_table.T)
    word_rows = _word_gather(wt_pad, word_ids.T.reshape(-1))
    word_lmaj = _word_to_batch_minor(word_rows, b, l, wd)
    word_out = jnp.transpose(word_lmaj, (2, 0, 1))

    char_lmaj = _char_lookup(char_table, jnp.transpose(char_ids, (1, 2, 0)))
    char_out = jnp.transpose(char_lmaj, (3, 0, 1, 2))
    return (word_out, char_out)
